# Initial kernel scaffold; baseline (speedup 1.0000x reference)
import jax
import jax.numpy as jnp
from jax.experimental import pallas as pl


def kernel(feats, batch_index, W1, b1, W2, b2):
    raise NotImplementedError("write your pallas kernel here")


# speedup vs baseline: 6.7312x; 6.7312x over previous
import functools

import jax
import jax.numpy as jnp
from jax import lax
from jax.experimental import pallas as pl
from jax.experimental.pallas import tpu as pltpu
from jax.experimental.pallas import tpu_sc as plsc

_N = 320000
_C = 128
_B = 32
_NC = 2
_NS = 16
_NW = _NC * _NS
_RPW = _N // _NW
_T = 200
_NT = _RPW // _T
_G = _C // 16


def _wid():
    return lax.axis_index("s") * _NC + lax.axis_index("c")


def _cnt_add(cnt, seg, amt):
    iota = lax.iota(jnp.int32, 16)
    for h in range(2):
        sel = jnp.where(iota == (seg - 16 * h), amt, 0.0)
        cnt[pl.ds(16 * h, 16)] = cnt[pl.ds(16 * h, 16)] + sel


def _p1_body(feats, idx, psum, pmax, pcnt, buf, idxb, accs, accm, cnt):
    wid = _wid()
    row0 = wid * _RPW

    zero = jnp.zeros((16,), jnp.float32)
    ninf = jnp.full((16,), -jnp.inf, jnp.float32)

    def init_body(i, _):
        accs[pl.ds(i * 16, 16)] = zero
        accm[pl.ds(i * 16, 16)] = ninf
        return 0

    lax.fori_loop(0, _B * _C // 16, init_body, 0)
    cnt[pl.ds(0, 16)] = zero
    cnt[pl.ds(16, 16)] = zero
    idxb[pl.ds(_T, 16)] = jnp.full((16,), 127, jnp.int32)

    def tile_body(t, _):
        r0 = row0 + t * _T
        pltpu.sync_copy(feats.at[pl.ds(r0, _T)], buf)
        pltpu.sync_copy(idx.at[pl.ds(r0, _T)], idxb.at[pl.ds(0, _T)])
        segf = idxb[pl.ds(0, 16)][0]
        segl = idxb[pl.ds(_T - 16, 16)][15]

        @pl.when(segf == segl)
        def _fast():
            def row_body(i, carry):
                sums, maxs = carry
                new_s, new_m = [], []
                for j in range(_G):
                    v = buf[i, pl.ds(16 * j, 16)]
                    new_s.append(sums[j] + v)
                    new_m.append(jnp.maximum(maxs[j], v))
                return (tuple(new_s), tuple(new_m))

            sums0 = tuple(zero for _ in range(_G))
            maxs0 = tuple(ninf for _ in range(_G))
            sums, maxs = lax.fori_loop(0, _T, row_body, (sums0, maxs0))
            base = segf * _C
            for j in range(_G):
                off = base + 16 * j
                accs[pl.ds(off, 16)] = accs[pl.ds(off, 16)] + sums[j]
                accm[pl.ds(off, 16)] = jnp.maximum(accm[pl.ds(off, 16)], maxs[j])
            _cnt_add(cnt, segf, float(_T))

        @pl.when(segf != segl)
        def _slow():
            def row_body(i, _):
                seg = idxb[pl.ds(i, 16)][0]
                base = seg * _C
                for j in range(_G):
                    off = base + 16 * j
                    v = buf[i, pl.ds(16 * j, 16)]
                    accs[pl.ds(off, 16)] = accs[pl.ds(off, 16)] + v
                    accm[pl.ds(off, 16)] = jnp.maximum(accm[pl.ds(off, 16)], v)
                _cnt_add(cnt, seg, 1.0)
                return 0

            lax.fori_loop(0, _T, row_body, 0)

        return 0

    lax.fori_loop(0, _NT, tile_body, 0)
    pltpu.sync_copy(accs, psum.at[wid])
    pltpu.sync_copy(accm, pmax.at[wid])
    pltpu.sync_copy(cnt, pcnt.at[wid])


def _p3_body(feats, idx, atts, out, buf, idxb, attb):
    wid = _wid()
    row0 = wid * _RPW
    pltpu.sync_copy(atts, attb)
    idxb[pl.ds(_T, 16)] = jnp.full((16,), 127, jnp.int32)

    def tile_body(t, _):
        r0 = row0 + t * _T
        pltpu.sync_copy(feats.at[pl.ds(r0, _T)], buf)
        pltpu.sync_copy(idx.at[pl.ds(r0, _T)], idxb.at[pl.ds(0, _T)])
        segf = idxb[pl.ds(0, 16)][0]
        segl = idxb[pl.ds(_T - 16, 16)][15]

        @pl.when(segf == segl)
        def _fast():
            base = segf * _C
            avecs = [attb[pl.ds(base + 16 * j, 16)] for j in range(_G)]

            def row_body(i, _):
                for j in range(_G):
                    buf[i, pl.ds(16 * j, 16)] = buf[i, pl.ds(16 * j, 16)] * avecs[j]
                return 0

            lax.fori_loop(0, _T, row_body, 0)

        @pl.when(segf != segl)
        def _slow():
            def row_body(i, _):
                seg = idxb[pl.ds(i, 16)][0]
                base = seg * _C
                for j in range(_G):
                    a = attb[pl.ds(base + 16 * j, 16)]
                    buf[i, pl.ds(16 * j, 16)] = buf[i, pl.ds(16 * j, 16)] * a
                return 0

            lax.fori_loop(0, _T, row_body, 0)

        pltpu.sync_copy(buf, out.at[pl.ds(r0, _T)])
        return 0

    lax.fori_loop(0, _NT, tile_body, 0)


@functools.lru_cache(maxsize=None)
def _build_sc_kernels():
    mesh = plsc.VectorSubcoreMesh(core_axis_name="c", subcore_axis_name="s")
    p1 = pl.kernel(
        _p1_body,
        out_type=(
            jax.ShapeDtypeStruct((_NW, _B * _C), jnp.float32),
            jax.ShapeDtypeStruct((_NW, _B * _C), jnp.float32),
            jax.ShapeDtypeStruct((_NW, _B), jnp.float32),
        ),
        mesh=mesh,
        scratch_types=[
            pltpu.VMEM((_T, _C), jnp.float32),
            pltpu.VMEM((_T + 16,), jnp.int32),
            pltpu.VMEM((_B * _C,), jnp.float32),
            pltpu.VMEM((_B * _C,), jnp.float32),
            pltpu.VMEM((_B,), jnp.float32),
        ],
        name="seg_pool_sc",
    )
    p3 = pl.kernel(
        _p3_body,
        out_type=jax.ShapeDtypeStruct((_N, _C), jnp.float32),
        mesh=mesh,
        scratch_types=[
            pltpu.VMEM((_T, _C), jnp.float32),
            pltpu.VMEM((_T + 16,), jnp.int32),
            pltpu.VMEM((_B * _C,), jnp.float32),
        ],
        name="scale_sc",
    )
    return p1, p3


def _mlp_tc(psum, pmax, pcnt, W1, b1, W2, b2):
    def body(ps, pm, pc, w1, b1r, w2, b2r, o):
        s = jnp.zeros((_B, _C), jnp.float32)
        m = jnp.full((_B, _C), -jnp.inf, jnp.float32)
        for w in range(_NW):
            s = s + ps[w * _B:(w + 1) * _B, :]
            m = jnp.maximum(m, pm[w * _B:(w + 1) * _B, :])
        c = jnp.sum(pc[...], axis=0)
        avgp = s / jnp.maximum(c, 1.0)[:, None]
        maxp = jnp.where(c[:, None] > 0, m, 0.0)

        def mlp(x):
            h = jnp.maximum(
                jnp.dot(x, w1[...], preferred_element_type=jnp.float32)
                + b1r[...], 0.0)
            return (jnp.dot(h, w2[...], preferred_element_type=jnp.float32)
                    + b2r[...])

        o[...] = jax.nn.sigmoid(mlp(avgp) + mlp(maxp))

    return pl.pallas_call(
        body,
        out_shape=jax.ShapeDtypeStruct((_B, _C), jnp.float32),
    )(psum.reshape(_NW * _B, _C), pmax.reshape(_NW * _B, _C), pcnt,
      W1, b1.reshape(1, -1), W2, b2.reshape(1, -1))


@jax.jit
def _impl(feats, idx, W1, b1, W2, b2):
    p1, p3 = _build_sc_kernels()
    psum, pmax, pcnt = p1(feats, idx)
    atts = _mlp_tc(psum, pmax, pcnt, W1, b1, W2, b2)
    return p3(feats, idx, atts.reshape(_B * _C))


def kernel(feats, batch_index, W1, b1, W2, b2):
    return _impl(feats, batch_index.astype(jnp.int32), W1, b1, W2, b2)


# speedup vs baseline: 12.1514x; 1.8052x over previous
import functools

import jax
import jax.numpy as jnp
from jax import lax
from jax.experimental import pallas as pl
from jax.experimental.pallas import tpu as pltpu
from jax.experimental.pallas import tpu_sc as plsc

_N = 320000
_C = 128
_B = 32
_NC = 2
_NS = 16
_NW = _NC * _NS
_RPW = _N // _NW
_T = 200
_NT = _RPW // _T
_G = _C // 16


def _wid():
    return lax.axis_index("s") * _NC + lax.axis_index("c")


def _cnt_add(cnt, seg, amt):
    iota = lax.iota(jnp.int32, 16)
    for h in range(2):
        sel = jnp.where(iota == (seg - 16 * h), amt, 0.0)
        cnt[pl.ds(16 * h, 16)] = cnt[pl.ds(16 * h, 16)] + sel


def _p1_body(feats, idx, psum, pmax, pcnt,
             ibuf0, ibuf1, idxb0, idxb1, accs, accm, cnt,
             in_sem0, in_sem1, idx_sem0, idx_sem1):
    ibuf = [ibuf0, ibuf1]
    idxb = [idxb0, idxb1]
    in_sem = [in_sem0, in_sem1]
    idx_sem = [idx_sem0, idx_sem1]
    wid = _wid()
    row0 = wid * _RPW

    zero = jnp.zeros((16,), jnp.float32)
    ninf = jnp.full((16,), -jnp.inf, jnp.float32)

    def init_body(i, _):
        accs[pl.ds(i * 16, 16)] = zero
        accm[pl.ds(i * 16, 16)] = ninf
        return 0

    lax.fori_loop(0, _B * _C // 16, init_body, 0)
    cnt[pl.ds(0, 16)] = zero
    cnt[pl.ds(16, 16)] = zero
    for b in range(2):
        idxb[b][pl.ds(_T, 16)] = jnp.full((16,), 127, jnp.int32)

    def start_in(t, b):
        r0 = row0 + t * _T
        pltpu.async_copy(feats.at[pl.ds(r0, _T)], ibuf[b], in_sem[b])
        pltpu.async_copy(idx.at[pl.ds(r0, _T)], idxb[b].at[pl.ds(0, _T)],
                         idx_sem[b])

    def wait_in(b):
        pltpu.make_async_copy(feats.at[pl.ds(0, _T)], ibuf[b],
                              in_sem[b]).wait()
        pltpu.make_async_copy(idx.at[pl.ds(0, _T)], idxb[b].at[pl.ds(0, _T)],
                              idx_sem[b]).wait()

    start_in(0, 0)

    def pair_body(t2, _):
        for b in range(2):
            t = 2 * t2 + b

            @pl.when(t + 1 < _NT)
            def _pref():
                start_in(t + 1, 1 - b)

            wait_in(b)
            buf = ibuf[b]
            idv = idxb[b]
            segf = idv[pl.ds(0, 16)][0]
            segl = idv[pl.ds(_T - 16, 16)][15]

            @pl.when(segf == segl)
            def _fast():
                def row_body(i, carry):
                    sums, maxs = carry
                    new_s, new_m = [], []
                    for j in range(_G):
                        v = buf[i, pl.ds(16 * j, 16)]
                        new_s.append(sums[j] + v)
                        new_m.append(jnp.maximum(maxs[j], v))
                    return (tuple(new_s), tuple(new_m))

                sums0 = tuple(zero for _ in range(_G))
                maxs0 = tuple(ninf for _ in range(_G))
                sums, maxs = lax.fori_loop(0, _T, row_body, (sums0, maxs0))
                base = segf * _C
                for j in range(_G):
                    off = base + 16 * j
                    accs[pl.ds(off, 16)] = accs[pl.ds(off, 16)] + sums[j]
                    accm[pl.ds(off, 16)] = jnp.maximum(accm[pl.ds(off, 16)],
                                                       maxs[j])
                _cnt_add(cnt, segf, float(_T))

            @pl.when(segf != segl)
            def _slow():
                def row_body(i, _):
                    seg = idv[pl.ds(i, 16)][0]
                    base = seg * _C
                    for j in range(_G):
                        off = base + 16 * j
                        v = buf[i, pl.ds(16 * j, 16)]
                        accs[pl.ds(off, 16)] = accs[pl.ds(off, 16)] + v
                        accm[pl.ds(off, 16)] = jnp.maximum(
                            accm[pl.ds(off, 16)], v)
                    _cnt_add(cnt, seg, 1.0)
                    return 0

                lax.fori_loop(0, _T, row_body, 0)

        return 0

    lax.fori_loop(0, _NT // 2, pair_body, 0)
    pltpu.sync_copy(accs, psum.at[wid])
    pltpu.sync_copy(accm, pmax.at[wid])
    pltpu.sync_copy(cnt, pcnt.at[wid])


def _p3_body(feats, idx, atts, out,
             ibuf0, ibuf1, obuf0, obuf1, idxb0, idxb1, attb,
             in_sem0, in_sem1, idx_sem0, idx_sem1, out_sem0, out_sem1):
    ibuf = [ibuf0, ibuf1]
    obuf = [obuf0, obuf1]
    idxb = [idxb0, idxb1]
    in_sem = [in_sem0, in_sem1]
    idx_sem = [idx_sem0, idx_sem1]
    out_sem = [out_sem0, out_sem1]
    wid = _wid()
    row0 = wid * _RPW
    pltpu.sync_copy(atts, attb)
    for b in range(2):
        idxb[b][pl.ds(_T, 16)] = jnp.full((16,), 127, jnp.int32)

    def start_in(t, b):
        r0 = row0 + t * _T
        pltpu.async_copy(feats.at[pl.ds(r0, _T)], ibuf[b], in_sem[b])
        pltpu.async_copy(idx.at[pl.ds(r0, _T)], idxb[b].at[pl.ds(0, _T)],
                         idx_sem[b])

    def wait_in(b):
        pltpu.make_async_copy(feats.at[pl.ds(0, _T)], ibuf[b],
                              in_sem[b]).wait()
        pltpu.make_async_copy(idx.at[pl.ds(0, _T)], idxb[b].at[pl.ds(0, _T)],
                              idx_sem[b]).wait()

    def wait_out(b):
        pltpu.make_async_copy(obuf[b], out.at[pl.ds(0, _T)],
                              out_sem[b]).wait()

    start_in(0, 0)

    def pair_body(t2, _):
        for b in range(2):
            t = 2 * t2 + b

            @pl.when(t + 1 < _NT)
            def _pref():
                start_in(t + 1, 1 - b)

            wait_in(b)

            @pl.when(t >= 2)
            def _wout():
                wait_out(b)

            buf = ibuf[b]
            ob = obuf[b]
            idv = idxb[b]
            segf = idv[pl.ds(0, 16)][0]
            segl = idv[pl.ds(_T - 16, 16)][15]

            @pl.when(segf == segl)
            def _fast():
                base = segf * _C
                avecs = [attb[pl.ds(base + 16 * j, 16)] for j in range(_G)]

                def row_body(i, _):
                    for j in range(_G):
                        ob[i, pl.ds(16 * j, 16)] = (
                            buf[i, pl.ds(16 * j, 16)] * avecs[j])
                    return 0

                lax.fori_loop(0, _T, row_body, 0)

            @pl.when(segf != segl)
            def _slow():
                def row_body(i, _):
                    seg = idv[pl.ds(i, 16)][0]
                    base = seg * _C
                    for j in range(_G):
                        a = attb[pl.ds(base + 16 * j, 16)]
                        ob[i, pl.ds(16 * j, 16)] = (
                            buf[i, pl.ds(16 * j, 16)] * a)
                    return 0

                lax.fori_loop(0, _T, row_body, 0)

            r0 = row0 + t * _T
            pltpu.async_copy(ob, out.at[pl.ds(r0, _T)], out_sem[b])

        return 0

    lax.fori_loop(0, _NT // 2, pair_body, 0)
    wait_out(0)
    wait_out(1)


@functools.lru_cache(maxsize=None)
def _build_sc_kernels():
    mesh = plsc.VectorSubcoreMesh(core_axis_name="c", subcore_axis_name="s")
    p1 = pl.kernel(
        _p1_body,
        out_type=(
            jax.ShapeDtypeStruct((_NW, _B * _C), jnp.float32),
            jax.ShapeDtypeStruct((_NW, _B * _C), jnp.float32),
            jax.ShapeDtypeStruct((_NW, _B), jnp.float32),
        ),
        mesh=mesh,
        scratch_types=[
            pltpu.VMEM((_T, _C), jnp.float32),
            pltpu.VMEM((_T, _C), jnp.float32),
            pltpu.VMEM((_T + 16,), jnp.int32),
            pltpu.VMEM((_T + 16,), jnp.int32),
            pltpu.VMEM((_B * _C,), jnp.float32),
            pltpu.VMEM((_B * _C,), jnp.float32),
            pltpu.VMEM((_B,), jnp.float32),
            pltpu.SemaphoreType.DMA,
            pltpu.SemaphoreType.DMA,
            pltpu.SemaphoreType.DMA,
            pltpu.SemaphoreType.DMA,
        ],
        name="seg_pool_sc",
    )
    p3 = pl.kernel(
        _p3_body,
        out_type=jax.ShapeDtypeStruct((_N, _C), jnp.float32),
        mesh=mesh,
        scratch_types=[
            pltpu.VMEM((_T, _C), jnp.float32),
            pltpu.VMEM((_T, _C), jnp.float32),
            pltpu.VMEM((_T, _C), jnp.float32),
            pltpu.VMEM((_T, _C), jnp.float32),
            pltpu.VMEM((_T + 16,), jnp.int32),
            pltpu.VMEM((_T + 16,), jnp.int32),
            pltpu.VMEM((_B * _C,), jnp.float32),
            pltpu.SemaphoreType.DMA,
            pltpu.SemaphoreType.DMA,
            pltpu.SemaphoreType.DMA,
            pltpu.SemaphoreType.DMA,
            pltpu.SemaphoreType.DMA,
            pltpu.SemaphoreType.DMA,
        ],
        name="scale_sc",
    )
    return p1, p3


def _mlp_tc(psum, pmax, pcnt, W1, b1, W2, b2):
    def body(ps, pm, pc, w1, b1r, w2, b2r, o):
        s = jnp.zeros((_B, _C), jnp.float32)
        m = jnp.full((_B, _C), -jnp.inf, jnp.float32)
        for w in range(_NW):
            s = s + ps[w * _B:(w + 1) * _B, :]
            m = jnp.maximum(m, pm[w * _B:(w + 1) * _B, :])
        c = jnp.sum(pc[...], axis=0)
        avgp = s / jnp.maximum(c, 1.0)[:, None]
        maxp = jnp.where(c[:, None] > 0, m, 0.0)

        def mlp(x):
            h = jnp.maximum(
                jnp.dot(x, w1[...], preferred_element_type=jnp.float32)
                + b1r[...], 0.0)
            return (jnp.dot(h, w2[...], preferred_element_type=jnp.float32)
                    + b2r[...])

        o[...] = jax.nn.sigmoid(mlp(avgp) + mlp(maxp))

    return pl.pallas_call(
        body,
        out_shape=jax.ShapeDtypeStruct((_B, _C), jnp.float32),
    )(psum.reshape(_NW * _B, _C), pmax.reshape(_NW * _B, _C), pcnt,
      W1, b1.reshape(1, -1), W2, b2.reshape(1, -1))


@jax.jit
def _impl(feats, idx, W1, b1, W2, b2):
    p1, p3 = _build_sc_kernels()
    psum, pmax, pcnt = p1(feats, idx)
    atts = _mlp_tc(psum, pmax, pcnt, W1, b1, W2, b2)
    return p3(feats, idx, atts.reshape(_B * _C))


def kernel(feats, batch_index, W1, b1, W2, b2):
    return _impl(feats, batch_index.astype(jnp.int32), W1, b1, W2, b2)
